# baseline (speedup 1.0000x reference)
import jax
import jax.numpy as jnp
from jax import lax
from jax.experimental import pallas as pl
from jax.experimental.pallas import tpu as pltpu
from jax.experimental.pallas import tpu_sc as plsc

B = 16384
F = 32
ROWS_PER_PHYS = 128 // F
NC = 2
NS = 16
NW = NC * NS
BPW = B // NW
NCHUNK = 4
CH = BPW // NCHUNK
L = 16


def _gmf_body(user_hbm, item_hbm, eu_hbm, ei_hbm, par_hbm, out_hbm,
              idx_u, idx_i, pid_u, pid_i, buf_u, buf_i, out_v, par_v,
              sem_u0, sem_u1, sem_i0, sem_i1):
    wid = lax.axis_index("s") * NC + lax.axis_index("c")
    base = wid * BPW

    for j in range(NCHUNK):
        pltpu.sync_copy(user_hbm.at[pl.ds(base + j * CH, CH)], idx_u.at[j])
        pltpu.sync_copy(item_hbm.at[pl.ds(base + j * CH, CH)], idx_i.at[j])
    pltpu.sync_copy(par_hbm, par_v)

    for j in range(NCHUNK):
        for k in range(CH // L):
            s = pl.ds(k * L, L)
            pid_u[j, s] = idx_u[j, s] >> 2
            pid_i[j, s] = idx_i[j, s] >> 2

    sems_u = (sem_u0, sem_u1)
    sems_i = (sem_i0, sem_i1)

    def fire(c):
        slot = c % 2
        return (pltpu.async_copy(eu_hbm.at[pid_u.at[c]], buf_u.at[slot], sems_u[slot]),
                pltpu.async_copy(ei_hbm.at[pid_i.at[c]], buf_i.at[slot], sems_i[slot]))

    w_lo = par_v[pl.ds(0, L)]
    w_hi = par_v[pl.ds(L, L)]
    bv = par_v[pl.ds(2 * L, L)]
    bsc = bv[0]
    lanes = lax.iota(jnp.int32, L)

    pend = fire(0)
    for c in range(NCHUNK):
        slot = c % 2
        nxt = fire(c + 1) if c + 1 < NCHUNK else None
        for cp in pend:
            cp.wait()

        def group(g, carry, slot=slot, c=c):
            iv_u = idx_u[c, pl.ds(g * L, L)]
            iv_i = idx_i[c, pl.ds(g * L, L)]
            ou_v = (iv_u & 3) * F
            oi_v = (iv_i & 3) * F
            acc = bv
            for i in range(L):
                r = g * L + i
                ou = ou_v[i]
                oi = oi_v[i]
                t = (buf_u[slot, r, pl.ds(ou, L)] * buf_i[slot, r, pl.ds(oi, L)] * w_lo
                     + buf_u[slot, r, pl.ds(ou + L, L)] * buf_i[slot, r, pl.ds(oi + L, L)] * w_hi)
                s = jnp.sum(t) + bsc
                acc = jnp.where(lanes == i, s, acc)
            out_v[pl.ds(c * CH + g * L, L)] = acc
            return carry

        lax.fori_loop(0, CH // L, group, 0)
        pend = nxt

    pltpu.sync_copy(out_v, out_hbm.at[pl.ds(base, BPW)])


@jax.jit
def kernel(user, item, embed_user, embed_item, W, b):
    mesh = plsc.VectorSubcoreMesh(core_axis_name="c", subcore_axis_name="s")
    n_phys = (embed_user.shape[0] * F) // 128
    kern = pl.kernel(
        _gmf_body,
        out_type=jax.ShapeDtypeStruct((B,), jnp.float32),
        mesh=mesh,
        compiler_params=pltpu.CompilerParams(needs_layout_passes=False),
        scratch_types=[
            pltpu.VMEM((NCHUNK, CH), jnp.int32),
            pltpu.VMEM((NCHUNK, CH), jnp.int32),
            pltpu.VMEM((NCHUNK, CH), jnp.int32),
            pltpu.VMEM((NCHUNK, CH), jnp.int32),
            pltpu.VMEM((2, CH, 128), jnp.float32),
            pltpu.VMEM((2, CH, 128), jnp.float32),
            pltpu.VMEM((BPW,), jnp.float32),
            pltpu.VMEM((128,), jnp.float32),
            pltpu.SemaphoreType.DMA,
            pltpu.SemaphoreType.DMA,
            pltpu.SemaphoreType.DMA,
            pltpu.SemaphoreType.DMA,
        ],
    )
    params = jnp.concatenate([
        W.reshape(F).astype(jnp.float32),
        jnp.full((L,), b[0], dtype=jnp.float32),
        jnp.zeros((128 - F - L,), dtype=jnp.float32),
    ])
    return kern(user.astype(jnp.int32), item.astype(jnp.int32),
                embed_user.reshape(n_phys, 128), embed_item.reshape(n_phys, 128),
                params)


# speedup vs baseline: 3.3753x; 3.3753x over previous
import jax
import jax.numpy as jnp
from jax import lax
from jax.experimental import pallas as pl
from jax.experimental.pallas import tpu as pltpu
from jax.experimental.pallas import tpu_sc as plsc

B = 16384
F = 32
NC = 2
NS = 16
NW = NC * NS
BPW = B // NW
GRP = 8
NROUND = BPW // GRP
L = 16


def _gmf_body(user_hbm, item_hbm, eut_hbm, eit_hbm, par_hbm, out_hbm,
              idx_u, idx_i, blk_u, blk_i, out_v, par_v, sem_u, sem_i):
    wid = lax.axis_index("s") * NC + lax.axis_index("c")
    base = wid * BPW

    pltpu.sync_copy(user_hbm.at[pl.ds(base, BPW)], idx_u.at[pl.ds(0, BPW)])
    pltpu.sync_copy(item_hbm.at[pl.ds(base, BPW)], idx_i.at[pl.ds(0, BPW)])
    pltpu.sync_copy(par_hbm, par_v)

    w_lo = par_v[pl.ds(0, L)]
    w_hi = par_v[pl.ds(L, L)]
    bv = par_v[pl.ds(2 * L, L)]
    bsc = bv[0]
    lanes = lax.iota(jnp.int32, L)
    rows_lo = lax.iota(jnp.int32, L)
    rows_hi = rows_lo + L

    def round_body(g, acc):
        iv_u = idx_u[pl.ds(g * GRP, L)]
        iv_i = idx_i[pl.ds(g * GRP, L)]
        for k in range(GRP):
            ou = pl.multiple_of((iv_u[k] >> 7) * 128, 128)
            oi = pl.multiple_of((iv_i[k] >> 7) * 128, 128)
            pltpu.async_copy(eut_hbm.at[:, pl.ds(ou, 128)], blk_u.at[k], sem_u)
            pltpu.async_copy(eit_hbm.at[:, pl.ds(oi, 128)], blk_i.at[k], sem_i)
        for k in range(GRP):
            ou = pl.multiple_of((iv_u[k] >> 7) * 128, 128)
            oi = pl.multiple_of((iv_i[k] >> 7) * 128, 128)
            pltpu.make_async_copy(eut_hbm.at[:, pl.ds(ou, 128)], blk_u.at[k], sem_u).wait()
            pltpu.make_async_copy(eit_hbm.at[:, pl.ds(oi, 128)], blk_i.at[k], sem_i).wait()
        half = (g % 2) * GRP
        for k in range(GRP):
            cu = jnp.full((L,), iv_u[k] & 127, dtype=jnp.int32)
            ci = jnp.full((L,), iv_i[k] & 127, dtype=jnp.int32)
            eu_lo = plsc.load_gather(blk_u.at[k], [rows_lo, cu])
            eu_hi = plsc.load_gather(blk_u.at[k], [rows_hi, cu])
            ei_lo = plsc.load_gather(blk_i.at[k], [rows_lo, ci])
            ei_hi = plsc.load_gather(blk_i.at[k], [rows_hi, ci])
            t = eu_lo * ei_lo * w_lo + eu_hi * ei_hi * w_hi
            s = jnp.sum(t) + bsc
            acc = jnp.where(lanes == half + k, s, acc)

        @pl.when(g % 2 == 1)
        def _():
            out_v[pl.ds((g - 1) * GRP, L)] = acc

        return acc

    lax.fori_loop(0, NROUND, round_body, bv)

    pltpu.sync_copy(out_v, out_hbm.at[pl.ds(base, BPW)])


@jax.jit
def kernel(user, item, embed_user, embed_item, W, b):
    mesh = plsc.VectorSubcoreMesh(core_axis_name="c", subcore_axis_name="s")
    kern = pl.kernel(
        _gmf_body,
        out_type=jax.ShapeDtypeStruct((B,), jnp.float32),
        mesh=mesh,
        compiler_params=pltpu.CompilerParams(needs_layout_passes=False),
        scratch_types=[
            pltpu.VMEM((BPW + L,), jnp.int32),
            pltpu.VMEM((BPW + L,), jnp.int32),
            pltpu.VMEM((GRP, F, 128), jnp.float32),
            pltpu.VMEM((GRP, F, 128), jnp.float32),
            pltpu.VMEM((BPW,), jnp.float32),
            pltpu.VMEM((128,), jnp.float32),
            pltpu.SemaphoreType.DMA,
            pltpu.SemaphoreType.DMA,
        ],
    )
    params = jnp.concatenate([
        W.reshape(F).astype(jnp.float32),
        jnp.full((L,), b[0], dtype=jnp.float32),
        jnp.zeros((128 - F - L,), dtype=jnp.float32),
    ])
    return kern(user.astype(jnp.int32), item.astype(jnp.int32),
                embed_user.T, embed_item.T, params)


# speedup vs baseline: 3.8347x; 1.1361x over previous
import jax
import jax.numpy as jnp
from jax import lax
from jax.experimental import pallas as pl
from jax.experimental.pallas import tpu as pltpu
from jax.experimental.pallas import tpu_sc as plsc

B = 16384
F = 32
NC = 2
NS = 16
NW = NC * NS
BPW = B // NW
GRP = 4
NROUND = BPW // GRP
L = 16


def _gmf_body(user_hbm, item_hbm, eut_hbm, eit_hbm, par_hbm, out_hbm,
              idx_u, idx_i, blk_u, blk_i, out_v, par_v,
              sem_u0, sem_u1, sem_i0, sem_i1):
    wid = lax.axis_index("s") * NC + lax.axis_index("c")
    base = wid * BPW

    pltpu.sync_copy(user_hbm.at[pl.ds(base, BPW)], idx_u.at[pl.ds(0, BPW)])
    pltpu.sync_copy(item_hbm.at[pl.ds(base, BPW)], idx_i.at[pl.ds(0, BPW)])
    pltpu.sync_copy(par_hbm, par_v)

    w_lo = par_v[pl.ds(0, L)]
    w_hi = par_v[pl.ds(L, L)]
    bv = par_v[pl.ds(2 * L, L)]
    bsc = bv[0]
    lanes = lax.iota(jnp.int32, L)
    rows_lo = lax.iota(jnp.int32, L)
    rows_hi = rows_lo + L
    sems = ((sem_u0, sem_i0), (sem_u1, sem_i1))

    def fire(g, slot):
        iv_u = idx_u[pl.ds(g * GRP, L)]
        iv_i = idx_i[pl.ds(g * GRP, L)]
        su, si = sems[slot]
        for k in range(GRP):
            ou = pl.multiple_of((iv_u[k] >> 7) * 128, 128)
            oi = pl.multiple_of((iv_i[k] >> 7) * 128, 128)
            pltpu.async_copy(eut_hbm.at[:, pl.ds(ou, 128)], blk_u.at[slot, k], su)
            pltpu.async_copy(eit_hbm.at[:, pl.ds(oi, 128)], blk_i.at[slot, k], si)

    def wait_and_compute(g, slot, acc):
        iv_u = idx_u[pl.ds(g * GRP, L)]
        iv_i = idx_i[pl.ds(g * GRP, L)]
        su, si = sems[slot]
        for k in range(GRP):
            ou = pl.multiple_of((iv_u[k] >> 7) * 128, 128)
            oi = pl.multiple_of((iv_i[k] >> 7) * 128, 128)
            pltpu.make_async_copy(eut_hbm.at[:, pl.ds(ou, 128)], blk_u.at[slot, k], su).wait()
            pltpu.make_async_copy(eit_hbm.at[:, pl.ds(oi, 128)], blk_i.at[slot, k], si).wait()
        lane0 = (g % 4) * GRP
        for k in range(GRP):
            cu = jnp.full((L,), iv_u[k] & 127, dtype=jnp.int32)
            ci = jnp.full((L,), iv_i[k] & 127, dtype=jnp.int32)
            eu_lo = plsc.load_gather(blk_u.at[slot, k], [rows_lo, cu])
            eu_hi = plsc.load_gather(blk_u.at[slot, k], [rows_hi, cu])
            ei_lo = plsc.load_gather(blk_i.at[slot, k], [rows_lo, ci])
            ei_hi = plsc.load_gather(blk_i.at[slot, k], [rows_hi, ci])
            t = eu_lo * ei_lo * w_lo + eu_hi * ei_hi * w_hi
            s = jnp.sum(t) + bsc
            acc = jnp.where(lanes == lane0 + k, s, acc)
        return acc

    def pair_body(h, acc):
        fire(2 * h + 1, 1)
        acc = wait_and_compute(2 * h, 0, acc)

        @pl.when(h < NROUND // 2 - 1)
        def _():
            fire(2 * h + 2, 0)

        acc = wait_and_compute(2 * h + 1, 1, acc)

        @pl.when(h % 2 == 1)
        def _():
            out_v[pl.ds((h // 2) * L, L)] = acc

        return acc

    fire(0, 0)
    lax.fori_loop(0, NROUND // 2, pair_body, bv)

    pltpu.sync_copy(out_v, out_hbm.at[pl.ds(base, BPW)])


@jax.jit
def kernel(user, item, embed_user, embed_item, W, b):
    mesh = plsc.VectorSubcoreMesh(core_axis_name="c", subcore_axis_name="s")
    kern = pl.kernel(
        _gmf_body,
        out_type=jax.ShapeDtypeStruct((B,), jnp.float32),
        mesh=mesh,
        compiler_params=pltpu.CompilerParams(needs_layout_passes=False),
        scratch_types=[
            pltpu.VMEM((BPW + L,), jnp.int32),
            pltpu.VMEM((BPW + L,), jnp.int32),
            pltpu.VMEM((2, GRP, F, 128), jnp.float32),
            pltpu.VMEM((2, GRP, F, 128), jnp.float32),
            pltpu.VMEM((BPW,), jnp.float32),
            pltpu.VMEM((128,), jnp.float32),
            pltpu.SemaphoreType.DMA,
            pltpu.SemaphoreType.DMA,
            pltpu.SemaphoreType.DMA,
            pltpu.SemaphoreType.DMA,
        ],
    )
    params = jnp.concatenate([
        W.reshape(F).astype(jnp.float32),
        jnp.full((L,), b[0], dtype=jnp.float32),
        jnp.zeros((128 - F - L,), dtype=jnp.float32),
    ])
    return kern(user.astype(jnp.int32), item.astype(jnp.int32),
                embed_user.T, embed_item.T, params)
